# Initial kernel scaffold; baseline (speedup 1.0000x reference)
import jax
import jax.numpy as jnp
from jax.experimental import pallas as pl


def kernel(embeddings, labels):
    raise NotImplementedError("write your pallas kernel here")


# speedup vs baseline: 483.8675x; 483.8675x over previous
import functools

import jax
import jax.numpy as jnp
from jax.experimental import pallas as pl


def _miner_block(emb_full_ref, emb_rows_ref, lab_row_ref, lab_col_ref,
                 pos_d_ref, neg_d_ref, pos_i_ref, neg_i_ref, *, block_rows):
    n, _ = emb_full_ref.shape
    i = pl.program_id(0)

    emb_full = emb_full_ref[...]
    sq_full = jnp.sum(emb_full * emb_full, axis=1, keepdims=True)
    en_full = emb_full * jax.lax.rsqrt(jnp.maximum(sq_full, 1e-30))

    emb_rows = emb_rows_ref[...]
    sq_rows = jnp.sum(emb_rows * emb_rows, axis=1, keepdims=True)
    en_rows = emb_rows * jax.lax.rsqrt(jnp.maximum(sq_rows, 1e-30))

    g = jax.lax.dot_general(en_rows, en_full, (((1,), (1,)), ((), ())),
                            preferred_element_type=jnp.float32,
                            precision=jax.lax.Precision.HIGHEST)
    dist = 1.0 - g

    lab_r = lab_row_ref[...]
    lab_c = lab_col_ref[...]
    col_ids = jax.lax.broadcasted_iota(jnp.int32, (block_rows, n), 1)
    row_ids = i * block_rows + jax.lax.broadcasted_iota(
        jnp.int32, (block_rows, n), 0)
    same = lab_r == lab_c
    pos_mask = same & (row_ids != col_ids)

    inf = jnp.float32(jnp.inf)
    pos_vals = jnp.where(pos_mask, dist, -inf)
    pos_max = jnp.max(pos_vals, axis=1, keepdims=True)
    pos_idx = jnp.min(jnp.where(pos_vals == pos_max, col_ids, n),
                      axis=1, keepdims=True)

    neg_vals = jnp.where(same, inf, dist)
    neg_min = jnp.min(neg_vals, axis=1, keepdims=True)
    neg_idx = jnp.min(jnp.where(neg_vals == neg_min, col_ids, n),
                      axis=1, keepdims=True)

    pos_d_ref[...] = pos_max
    neg_d_ref[...] = neg_min
    pos_i_ref[...] = pos_idx
    neg_i_ref[...] = neg_idx


def kernel(embeddings, labels):
    n, d = embeddings.shape
    block_rows = 256
    grid = (n // block_rows,)
    lab_col = labels.reshape(1, n)
    lab_row = labels.reshape(n, 1)

    out_shapes = (
        jax.ShapeDtypeStruct((n, 1), jnp.float32),
        jax.ShapeDtypeStruct((n, 1), jnp.float32),
        jax.ShapeDtypeStruct((n, 1), jnp.int32),
        jax.ShapeDtypeStruct((n, 1), jnp.int32),
    )
    in_specs = [
        pl.BlockSpec((n, d), lambda i: (0, 0)),
        pl.BlockSpec((block_rows, d), lambda i: (i, 0)),
        pl.BlockSpec((block_rows, 1), lambda i: (i, 0)),
        pl.BlockSpec((1, n), lambda i: (0, 0)),
    ]
    out_specs = tuple(pl.BlockSpec((block_rows, 1), lambda i: (i, 0))
                      for _ in range(4))

    pos_d, neg_d, pos_i, neg_i = pl.pallas_call(
        functools.partial(_miner_block, block_rows=block_rows),
        grid=grid,
        in_specs=in_specs,
        out_specs=out_specs,
        out_shape=out_shapes,
    )(embeddings, embeddings, lab_row, lab_col)

    anchors = jnp.arange(n, dtype=jnp.int32)
    triplets = jnp.column_stack((anchors, pos_i[:, 0], neg_i[:, 0]))
    return (triplets, pos_d[:, 0], neg_d[:, 0])


# speedup vs baseline: 507.3183x; 1.0485x over previous
import functools

import jax
import jax.numpy as jnp
from jax.experimental import pallas as pl
from jax.experimental.pallas import tpu as pltpu


def _miner_block(emb_full_ref, lab_row_ref, lab_col_ref,
                 trip_ref, pos_d_ref, neg_d_ref, en_ref, *, block_rows):
    n, _ = emb_full_ref.shape
    i = pl.program_id(0)

    @pl.when(i == 0)
    def _():
        emb_full = emb_full_ref[...]
        sq_full = jnp.sum(emb_full * emb_full, axis=1, keepdims=True)
        en_ref[...] = emb_full * jax.lax.rsqrt(jnp.maximum(sq_full, 1e-30))

    en_full = en_ref[...]
    en_rows = en_ref[pl.ds(i * block_rows, block_rows), :]

    g = jax.lax.dot_general(en_rows, en_full, (((1,), (1,)), ((), ())),
                            preferred_element_type=jnp.float32,
                            precision=jax.lax.Precision.HIGHEST)
    dist = 1.0 - g

    lab_r = lab_row_ref[...]
    lab_c = lab_col_ref[...]
    col_ids = jax.lax.broadcasted_iota(jnp.int32, (block_rows, n), 1)
    row_ids = i * block_rows + jax.lax.broadcasted_iota(
        jnp.int32, (block_rows, n), 0)
    same = lab_r == lab_c
    pos_mask = same & (row_ids != col_ids)

    inf = jnp.float32(jnp.inf)
    pos_vals = jnp.where(pos_mask, dist, -inf)
    pos_max = jnp.max(pos_vals, axis=1, keepdims=True)
    pos_idx = jnp.min(jnp.where(pos_vals == pos_max, col_ids, n),
                      axis=1, keepdims=True)

    neg_vals = jnp.where(same, inf, dist)
    neg_min = jnp.min(neg_vals, axis=1, keepdims=True)
    neg_idx = jnp.min(jnp.where(neg_vals == neg_min, col_ids, n),
                      axis=1, keepdims=True)

    anchors = i * block_rows + jax.lax.broadcasted_iota(
        jnp.int32, (block_rows, 1), 0)
    trip_ref[...] = jnp.concatenate((anchors, pos_idx, neg_idx), axis=1)
    pos_d_ref[...] = pos_max
    neg_d_ref[...] = neg_min


def kernel(embeddings, labels):
    n, d = embeddings.shape
    block_rows = 256
    grid = (n // block_rows,)
    lab_col = labels.reshape(1, n)
    lab_row = labels.reshape(n, 1)

    out_shapes = (
        jax.ShapeDtypeStruct((n, 3), jnp.int32),
        jax.ShapeDtypeStruct((n, 1), jnp.float32),
        jax.ShapeDtypeStruct((n, 1), jnp.float32),
    )
    in_specs = [
        pl.BlockSpec((n, d), lambda i: (0, 0)),
        pl.BlockSpec((block_rows, 1), lambda i: (i, 0)),
        pl.BlockSpec((1, n), lambda i: (0, 0)),
    ]
    out_specs = (
        pl.BlockSpec((block_rows, 3), lambda i: (i, 0)),
        pl.BlockSpec((block_rows, 1), lambda i: (i, 0)),
        pl.BlockSpec((block_rows, 1), lambda i: (i, 0)),
    )

    triplets, pos_d, neg_d = pl.pallas_call(
        functools.partial(_miner_block, block_rows=block_rows),
        grid=grid,
        in_specs=in_specs,
        out_specs=out_specs,
        out_shape=out_shapes,
        scratch_shapes=[pltpu.VMEM((n, d), jnp.float32)],
    )(embeddings, lab_row, lab_col)

    return (triplets, pos_d[:, 0], neg_d[:, 0])


# speedup vs baseline: 547.0304x; 1.0783x over previous
import functools

import jax
import jax.numpy as jnp
from jax.experimental import pallas as pl
from jax.experimental.pallas import tpu as pltpu


def _miner_block(emb_full_ref, lab_row_ref, lab_col_ref,
                 trip_ref, pos_d_ref, neg_d_ref, en_ref, *, block_rows):
    n, _ = emb_full_ref.shape
    i = pl.program_id(0)

    @pl.when(i == 0)
    def _():
        emb_full = emb_full_ref[...]
        sq_full = jnp.sum(emb_full * emb_full, axis=1, keepdims=True)
        en_ref[...] = emb_full * jax.lax.rsqrt(jnp.maximum(sq_full, 1e-30))

    en_full = en_ref[...]
    en_rows = en_ref[pl.ds(i * block_rows, block_rows), :]

    g = jax.lax.dot_general(en_rows, en_full, (((1,), (1,)), ((), ())),
                            preferred_element_type=jnp.float32,
                            precision=jax.lax.Precision.HIGHEST)
    dist = 1.0 - g

    lab_r = lab_row_ref[...]
    lab_c = lab_col_ref[...]
    col_ids = jax.lax.broadcasted_iota(jnp.int32, (block_rows, n), 1)
    row_ids = i * block_rows + jax.lax.broadcasted_iota(
        jnp.int32, (block_rows, n), 0)
    same = lab_r == lab_c
    pos_mask = same & (row_ids != col_ids)

    inf = jnp.float32(jnp.inf)
    pos_vals = jnp.where(pos_mask, dist, -inf)
    pos_max = jnp.max(pos_vals, axis=1, keepdims=True)
    pos_idx = jnp.min(jnp.where(pos_vals == pos_max, col_ids, n),
                      axis=1, keepdims=True)

    neg_vals = jnp.where(same, inf, dist)
    neg_min = jnp.min(neg_vals, axis=1, keepdims=True)
    neg_idx = jnp.min(jnp.where(neg_vals == neg_min, col_ids, n),
                      axis=1, keepdims=True)

    anchors = i * block_rows + jax.lax.broadcasted_iota(
        jnp.int32, (block_rows, 1), 0)
    trip_ref[...] = jnp.concatenate((anchors, pos_idx, neg_idx), axis=1)
    pos_d_ref[...] = pos_max
    neg_d_ref[...] = neg_min


def kernel(embeddings, labels):
    n, d = embeddings.shape
    block_rows = 1024
    grid = (n // block_rows,)
    lab_col = labels.reshape(1, n)
    lab_row = labels.reshape(n, 1)

    out_shapes = (
        jax.ShapeDtypeStruct((n, 3), jnp.int32),
        jax.ShapeDtypeStruct((n, 1), jnp.float32),
        jax.ShapeDtypeStruct((n, 1), jnp.float32),
    )
    in_specs = [
        pl.BlockSpec((n, d), lambda i: (0, 0)),
        pl.BlockSpec((block_rows, 1), lambda i: (i, 0)),
        pl.BlockSpec((1, n), lambda i: (0, 0)),
    ]
    out_specs = (
        pl.BlockSpec((block_rows, 3), lambda i: (i, 0)),
        pl.BlockSpec((block_rows, 1), lambda i: (i, 0)),
        pl.BlockSpec((block_rows, 1), lambda i: (i, 0)),
    )

    triplets, pos_d, neg_d = pl.pallas_call(
        functools.partial(_miner_block, block_rows=block_rows),
        grid=grid,
        in_specs=in_specs,
        out_specs=out_specs,
        out_shape=out_shapes,
        scratch_shapes=[pltpu.VMEM((n, d), jnp.float32)],
    )(embeddings, lab_row, lab_col)

    return (triplets, pos_d[:, 0], neg_d[:, 0])
